# baseline (speedup 1.0000x reference)
import functools

import jax
import jax.numpy as jnp
from jax import lax
from jax.experimental import pallas as pl
from jax.experimental.pallas import tpu as pltpu
from jax.experimental.pallas import tpu_sc as plsc

_NC = 2
_NS = 16
_NW = _NC * _NS
_GRP = 128


def _mesh():
    return plsc.VectorSubcoreMesh(core_axis_name="c", subcore_axis_name="s")


_SC_PARAMS = pltpu.CompilerParams(use_tc_tiling_on_sc=False)


@functools.lru_cache(maxsize=None)
def _make_deg(n, n_pad, g_per_tile):
    f = 16
    rows_slice = n_pad // _NS

    @functools.partial(
        pl.kernel,
        mesh=_mesh(),
        out_type=jax.ShapeDtypeStruct((_NC, n_pad, f), jnp.float32),
        compiler_params=_SC_PARAMS,
        scratch_types=[
            pltpu.VMEM_SHARED((n_pad, f), jnp.float32),
            pltpu.VMEM((g_per_tile, _GRP), jnp.int32),
            pltpu.VMEM((_GRP, f), jnp.float32),
            pltpu.VMEM((rows_slice, f), jnp.float32),
            pltpu.SemaphoreType.DMA,
        ],
    )
    def deg_kernel(dst_hbm, out_hbm, acc_s, dst_v, ones_v, tmp_v, sem):
        c = lax.axis_index("c")
        s = lax.axis_index("s")
        wid = c * _NS + s

        def fill(i, _):
            tmp_v[i, :] = jnp.zeros((16,), jnp.float32)
            return 0

        lax.fori_loop(0, rows_slice, fill, 0)

        def fill_ones(i, _):
            ones_v[i, :] = jnp.ones((16,), jnp.float32)
            return 0

        lax.fori_loop(0, _GRP, fill_ones, 0)
        pltpu.sync_copy(tmp_v, acc_s.at[pl.ds(s * rows_slice, rows_slice)])
        plsc.subcore_barrier()

        pltpu.sync_copy(dst_hbm.at[wid], dst_v)

        def body(j, _):
            cps = [pltpu.async_copy(ones_v, acc_s.at[dst_v.at[4 * j + k]],
                                    sem, add=True) for k in range(4)]
            for cp in cps:
                cp.wait()
            return 0

        lax.fori_loop(0, g_per_tile // 4, body, 0)
        plsc.subcore_barrier()
        pltpu.sync_copy(
            acc_s.at[pl.ds(s * rows_slice, rows_slice)],
            out_hbm.at[c, pl.ds(s * rows_slice, rows_slice)],
        )

    return deg_kernel


@functools.lru_cache(maxsize=None)
def _make_agg(n, n_pad, g_per_tile, f):
    rows_slice = n_pad // _NS

    @functools.partial(
        pl.kernel,
        mesh=_mesh(),
        out_type=jax.ShapeDtypeStruct((_NC, n_pad, f), jnp.float32),
        compiler_params=_SC_PARAMS,
        scratch_types=[
            pltpu.VMEM_SHARED((n_pad, f), jnp.float32),
            pltpu.VMEM((g_per_tile, _GRP), jnp.int32),
            pltpu.VMEM((g_per_tile, _GRP), jnp.int32),
            pltpu.VMEM((_GRP, f), jnp.float32),
            pltpu.VMEM((_GRP, f), jnp.float32),
            pltpu.VMEM((rows_slice, f), jnp.float32),
            pltpu.SemaphoreType.DMA,
            pltpu.SemaphoreType.DMA,
            pltpu.SemaphoreType.DMA,
            pltpu.SemaphoreType.DMA,
        ],
    )
    def agg_kernel(y_hbm, src_hbm, dst_hbm, out_hbm,
                   acc_s, src_v, dst_v, rows_a, rows_b, tmp_v,
                   gsem_a, gsem_b, ssem_a, ssem_b):
        c = lax.axis_index("c")
        s = lax.axis_index("s")
        wid = c * _NS + s

        def fill(i, _):
            for j in range(f // 16):
                tmp_v[i, pl.ds(j * 16, 16)] = jnp.zeros((16,), jnp.float32)
            return 0

        lax.fori_loop(0, rows_slice, fill, 0)
        pltpu.sync_copy(tmp_v, acc_s.at[pl.ds(s * rows_slice, rows_slice)])
        pltpu.sync_copy(src_hbm.at[wid], src_v)
        pltpu.sync_copy(dst_hbm.at[wid], dst_v)
        plsc.subcore_barrier()

        n_pairs = g_per_tile // 2

        def gather(g, buf, sem):
            return pltpu.async_copy(y_hbm.at[src_v.at[g]], buf, sem)

        def scatter(g, buf, sem):
            return pltpu.async_copy(buf, acc_s.at[dst_v.at[g]], sem, add=True)

        gather(0, rows_a, gsem_a)

        def body(j, _):
            g0 = 2 * j
            pltpu.make_async_copy(y_hbm.at[src_v.at[g0]], rows_a,
                                  gsem_a).wait()

            @pl.when(j > 0)
            def _():
                pltpu.make_async_copy(rows_b, acc_s.at[dst_v.at[g0]],
                                      ssem_b).wait()

            gather(g0 + 1, rows_b, gsem_b)
            scatter(g0, rows_a, ssem_a)
            pltpu.make_async_copy(y_hbm.at[src_v.at[g0]], rows_b,
                                  gsem_b).wait()
            pltpu.make_async_copy(rows_a, acc_s.at[dst_v.at[g0]],
                                  ssem_a).wait()

            @pl.when(j + 1 < n_pairs)
            def _():
                gather(g0 + 2, rows_a, gsem_a)

            scatter(g0 + 1, rows_b, ssem_b)
            return 0

        lax.fori_loop(0, n_pairs, body, 0)
        pltpu.make_async_copy(rows_b, acc_s.at[dst_v.at[0]], ssem_b).wait()
        plsc.subcore_barrier()
        pltpu.sync_copy(
            acc_s.at[pl.ds(s * rows_slice, rows_slice)],
            out_hbm.at[c, pl.ds(s * rows_slice, rows_slice)],
        )

    return agg_kernel


def _tc_call(body, out_shapes, *args):
    return pl.pallas_call(body, out_shape=out_shapes)(*args)


def _tc1_body(x_ref, w1_ref, p0_ref, p1_ref, y1_ref, dinv_ref):
    n = x_ref.shape[0]
    pad = y1_ref.shape[0] - n
    deg = p0_ref[...] + p1_ref[...] + 1.0
    dinv = lax.rsqrt(deg)
    hw = jnp.dot(x_ref[...], w1_ref[...], preferred_element_type=jnp.float32)
    y1_ref[:n, :] = hw * dinv
    y1_ref[n:, :] = jnp.zeros((pad, y1_ref.shape[1]), jnp.float32)
    dinv_ref[...] = dinv


def _tc_mid_body(a0_ref, a1_ref, y_ref, dinv_ref, b_ref, g_ref, be_ref,
                 w_ref, out_ref):
    dinv = dinv_ref[...]
    z = (a0_ref[...] + a1_ref[...] + y_ref[...]) * dinv + b_ref[...]
    mu = jnp.mean(z, axis=0, keepdims=True)
    var = jnp.mean((z - mu) ** 2, axis=0, keepdims=True)
    h = (z - mu) * lax.rsqrt(var + 1e-5) * g_ref[...] + be_ref[...]
    h = jax.nn.gelu(h)
    n = a0_ref.shape[0]
    pad = out_ref.shape[0] - n
    out_ref[:n, :] = jnp.dot(h, w_ref[...],
                             preferred_element_type=jnp.float32) * dinv
    out_ref[n:, :] = jnp.zeros((pad, out_ref.shape[1]), jnp.float32)


def _tc_out_body(a0_ref, a1_ref, y_ref, dinv_ref, b_ref, out_ref):
    z = (a0_ref[...] + a1_ref[...] + y_ref[...]) * dinv_ref[...] + b_ref[...]
    m = jnp.max(z, axis=1, keepdims=True)
    sh = z - m
    out_ref[...] = sh - jnp.log(jnp.sum(jnp.exp(sh), axis=1, keepdims=True))


def kernel(x, edge_index, W1, b1, g1, be1, W2, b2, g2, be2, W3, b3):
    n, d = x.shape
    e = edge_index.shape[1]
    f1, f2, f3 = W1.shape[1], W2.shape[1], W3.shape[1]

    n_pad = (n + 1 + 127) // 128 * 128
    per_tile = (e + _NW - 1) // _NW
    g_per_tile = (per_tile + _GRP - 1) // _GRP
    g_per_tile = (g_per_tile + 3) // 4 * 4
    cap = _NW * g_per_tile * _GRP

    src = edge_index[0]
    dst = edge_index[1]
    pad = cap - e
    src_p = jnp.concatenate([src, jnp.zeros((pad,), jnp.int32)])
    dst_p = jnp.concatenate([dst, jnp.full((pad,), n, jnp.int32)])
    src_r = src_p.reshape(_NW, g_per_tile, _GRP)
    dst_r = dst_p.reshape(_NW, g_per_tile, _GRP)

    degp = _make_deg(n, n_pad, g_per_tile)(dst_r)
    y1, dinv = _tc_call(
        _tc1_body,
        [jax.ShapeDtypeStruct((n_pad, f1), jnp.float32),
         jax.ShapeDtypeStruct((n, 1), jnp.float32)],
        x, W1, degp[0, :n, :1], degp[1, :n, :1])

    agg16 = _make_agg(n, n_pad, g_per_tile, f1)
    agg32 = _make_agg(n, n_pad, g_per_tile, f2)

    a1 = agg16(y1, src_r, dst_r)
    y2 = _tc_call(
        _tc_mid_body, jax.ShapeDtypeStruct((n_pad, f2), jnp.float32),
        a1[0, :n], a1[1, :n], y1[:n], dinv,
        b1.reshape(1, f1), g1.reshape(1, f1), be1.reshape(1, f1), W2)

    a2 = agg32(y2, src_r, dst_r)
    y3 = _tc_call(
        _tc_mid_body, jax.ShapeDtypeStruct((n_pad, f3), jnp.float32),
        a2[0, :n], a2[1, :n], y2[:n], dinv,
        b2.reshape(1, f2), g2.reshape(1, f2), be2.reshape(1, f2), W3)

    a3 = _make_agg(n, n_pad, g_per_tile, f3)(y3, src_r, dst_r)
    out = _tc_call(
        _tc_out_body, jax.ShapeDtypeStruct((n, f3), jnp.float32),
        a3[0, :n], a3[1, :n], y3[:n], dinv, b3.reshape(1, f3))
    return out


# speedup vs baseline: 2.0306x; 2.0306x over previous
import functools

import jax
import jax.numpy as jnp
from jax import lax
from jax.experimental import pallas as pl
from jax.experimental.pallas import tpu as pltpu
from jax.experimental.pallas import tpu_sc as plsc

_NC = 2
_NS = 16
_NW = _NC * _NS
_GRP = 128


def _mesh():
    return plsc.VectorSubcoreMesh(core_axis_name="c", subcore_axis_name="s")


_SC_PARAMS = pltpu.CompilerParams(use_tc_tiling_on_sc=False)


@functools.lru_cache(maxsize=None)
def _make_deg(n, n_pad, g_per_tile):
    f = 8
    rows_slice = n_pad // _NS

    @functools.partial(
        pl.kernel,
        mesh=_mesh(),
        out_type=jax.ShapeDtypeStruct((_NC, n_pad, f), jnp.float32),
        compiler_params=_SC_PARAMS,
        scratch_types=[
            pltpu.VMEM_SHARED((n_pad, f), jnp.float32),
            pltpu.VMEM((g_per_tile, _GRP), jnp.int32),
            pltpu.VMEM((_GRP, f), jnp.float32),
            pltpu.SemaphoreType.DMA,
        ],
    )
    def deg_kernel(dst_hbm, ones_hbm, zeros_hbm, out_hbm,
                   acc_s, dst_v, ones_v, sem):
        c = lax.axis_index("c")
        s = lax.axis_index("s")
        wid = c * _NS + s

        pltpu.sync_copy(ones_hbm, ones_v)
        pltpu.sync_copy(zeros_hbm,
                        acc_s.at[pl.ds(s * rows_slice, rows_slice)])
        plsc.subcore_barrier()

        pltpu.sync_copy(dst_hbm.at[wid], dst_v)

        def body(j, _):
            cps = [pltpu.async_copy(ones_v, acc_s.at[dst_v.at[4 * j + k]],
                                    sem, add=True) for k in range(4)]
            for cp in cps:
                cp.wait()
            return 0

        lax.fori_loop(0, g_per_tile // 4, body, 0)
        plsc.subcore_barrier()
        pltpu.sync_copy(
            acc_s.at[pl.ds(s * rows_slice, rows_slice)],
            out_hbm.at[c, pl.ds(s * rows_slice, rows_slice)],
        )

    return deg_kernel


@functools.lru_cache(maxsize=None)
def _make_agg(n, n_pad, g_per_tile, f):
    rows_slice = n_pad // _NS

    @functools.partial(
        pl.kernel,
        mesh=_mesh(),
        out_type=jax.ShapeDtypeStruct((_NC, n_pad, f), jnp.float32),
        compiler_params=_SC_PARAMS,
        scratch_types=[
            pltpu.VMEM_SHARED((n_pad, f), jnp.float32),
            pltpu.VMEM_SHARED((n_pad, f), jnp.float32),
            pltpu.VMEM((g_per_tile, _GRP), jnp.int32),
            pltpu.VMEM((g_per_tile, _GRP), jnp.int32),
            pltpu.VMEM((_GRP, f), jnp.float32),
            pltpu.VMEM((_GRP, f), jnp.float32),
            pltpu.VMEM((_GRP, f), jnp.float32),
            pltpu.VMEM((_GRP, f), jnp.float32),
            pltpu.VMEM((rows_slice, f), jnp.float32),
            pltpu.SemaphoreType.DMA,
            pltpu.SemaphoreType.DMA,
            pltpu.SemaphoreType.DMA,
            pltpu.SemaphoreType.DMA,
            pltpu.SemaphoreType.DMA,
            pltpu.SemaphoreType.DMA,
            pltpu.SemaphoreType.DMA,
            pltpu.SemaphoreType.DMA,
        ],
    )
    def agg_kernel(y_hbm, src_hbm, dst_hbm, out_hbm,
                   y_s, acc_s, src_v, dst_v, rows_a, rows_b, rows_c, rows_d,
                   tmp_v, gsem_a, gsem_b, gsem_c, gsem_d,
                   ssem_a, ssem_b, ssem_c, ssem_d):
        c = lax.axis_index("c")
        s = lax.axis_index("s")
        wid = c * _NS + s

        def fill(i, _):
            for j in range(f // 16):
                tmp_v[i, pl.ds(j * 16, 16)] = jnp.zeros((16,), jnp.float32)
            return 0

        lax.fori_loop(0, rows_slice, fill, 0)
        pltpu.sync_copy(tmp_v, acc_s.at[pl.ds(s * rows_slice, rows_slice)])
        pltpu.sync_copy(y_hbm.at[pl.ds(s * rows_slice, rows_slice)],
                        y_s.at[pl.ds(s * rows_slice, rows_slice)])
        pltpu.sync_copy(src_hbm.at[wid], src_v)
        pltpu.sync_copy(dst_hbm.at[wid], dst_v)
        plsc.subcore_barrier()

        bufs = (rows_a, rows_b, rows_c, rows_d)
        gsems = (gsem_a, gsem_b, gsem_c, gsem_d)
        ssems = (ssem_a, ssem_b, ssem_c, ssem_d)
        n_rounds = g_per_tile // 4

        def gather(g, k):
            pltpu.async_copy(y_s.at[src_v.at[g]], bufs[k], gsems[k])

        for k in range(4):
            gather(k, k)

        def body(j, _):
            for k in range(4):
                pltpu.make_async_copy(y_s.at[src_v.at[0]], bufs[k],
                                      gsems[k]).wait()
                pltpu.async_copy(bufs[k], acc_s.at[dst_v.at[4 * j + k]],
                                 ssems[k], add=True)
            for k in range(4):
                pltpu.make_async_copy(bufs[k], acc_s.at[dst_v.at[0]],
                                      ssems[k]).wait()

                @pl.when(j + 1 < n_rounds)
                def _():
                    gather(4 * (j + 1) + k, k)

            return 0

        lax.fori_loop(0, n_rounds, body, 0)
        plsc.subcore_barrier()
        pltpu.sync_copy(
            acc_s.at[pl.ds(s * rows_slice, rows_slice)],
            out_hbm.at[c, pl.ds(s * rows_slice, rows_slice)],
        )

    return agg_kernel


def _tc_call(body, out_shapes, *args):
    return pl.pallas_call(body, out_shape=out_shapes)(*args)


def _tc1_body(x_ref, w1_ref, p0_ref, p1_ref, y1_ref, dinv_ref):
    n = x_ref.shape[0]
    pad = y1_ref.shape[0] - n
    deg = p0_ref[...] + p1_ref[...] + 1.0
    dinv = lax.rsqrt(deg)
    hw = jnp.dot(x_ref[...], w1_ref[...], preferred_element_type=jnp.float32)
    y1_ref[:n, :] = hw * dinv
    y1_ref[n:, :] = jnp.zeros((pad, y1_ref.shape[1]), jnp.float32)
    dinv_ref[...] = dinv


def _tc_mid_body(a0_ref, a1_ref, y_ref, dinv_ref, b_ref, g_ref, be_ref,
                 w_ref, out_ref):
    dinv = dinv_ref[...]
    z = (a0_ref[...] + a1_ref[...] + y_ref[...]) * dinv + b_ref[...]
    mu = jnp.mean(z, axis=0, keepdims=True)
    var = jnp.mean((z - mu) ** 2, axis=0, keepdims=True)
    h = (z - mu) * lax.rsqrt(var + 1e-5) * g_ref[...] + be_ref[...]
    h = jax.nn.gelu(h)
    n = a0_ref.shape[0]
    pad = out_ref.shape[0] - n
    out_ref[:n, :] = jnp.dot(h, w_ref[...],
                             preferred_element_type=jnp.float32) * dinv
    out_ref[n:, :] = jnp.zeros((pad, out_ref.shape[1]), jnp.float32)


def _tc_out_body(a0_ref, a1_ref, y_ref, dinv_ref, b_ref, out_ref):
    z = (a0_ref[...] + a1_ref[...] + y_ref[...]) * dinv_ref[...] + b_ref[...]
    m = jnp.max(z, axis=1, keepdims=True)
    sh = z - m
    out_ref[...] = sh - jnp.log(jnp.sum(jnp.exp(sh), axis=1, keepdims=True))


def kernel(x, edge_index, W1, b1, g1, be1, W2, b2, g2, be2, W3, b3):
    n, d = x.shape
    e = edge_index.shape[1]
    f1, f2, f3 = W1.shape[1], W2.shape[1], W3.shape[1]

    n_pad = (n + 1 + 127) // 128 * 128
    per_tile = (e + _NW - 1) // _NW
    g_per_tile = (per_tile + _GRP - 1) // _GRP
    g_per_tile = (g_per_tile + 3) // 4 * 4
    cap = _NW * g_per_tile * _GRP

    src = edge_index[0]
    dst = edge_index[1]
    pad = cap - e
    src_p = jnp.concatenate([src, jnp.zeros((pad,), jnp.int32)])
    dst_p = jnp.concatenate([dst, jnp.full((pad,), n, jnp.int32)])
    src_r = src_p.reshape(_NW, g_per_tile, _GRP)
    dst_r = dst_p.reshape(_NW, g_per_tile, _GRP)

    ones8 = jnp.ones((_GRP, 8), jnp.float32)
    zeros8 = jnp.zeros((n_pad // _NS, 8), jnp.float32)
    degp = _make_deg(n, n_pad, g_per_tile)(dst_r, ones8, zeros8)
    y1, dinv = _tc_call(
        _tc1_body,
        [jax.ShapeDtypeStruct((n_pad, f1), jnp.float32),
         jax.ShapeDtypeStruct((n, 1), jnp.float32)],
        x, W1, degp[0, :n, :1], degp[1, :n, :1])

    agg16 = _make_agg(n, n_pad, g_per_tile, f1)
    agg32 = _make_agg(n, n_pad, g_per_tile, f2)

    a1 = agg16(y1, src_r, dst_r)
    y2 = _tc_call(
        _tc_mid_body, jax.ShapeDtypeStruct((n_pad, f2), jnp.float32),
        a1[0, :n], a1[1, :n], y1[:n], dinv,
        b1.reshape(1, f1), g1.reshape(1, f1), be1.reshape(1, f1), W2)

    a2 = agg32(y2, src_r, dst_r)
    y3 = _tc_call(
        _tc_mid_body, jax.ShapeDtypeStruct((n_pad, f3), jnp.float32),
        a2[0, :n], a2[1, :n], y2[:n], dinv,
        b2.reshape(1, f2), g2.reshape(1, f2), be2.reshape(1, f2), W3)

    a3 = _make_agg(n, n_pad, g_per_tile, f3)(y3, src_r, dst_r)
    out = _tc_call(
        _tc_out_body, jax.ShapeDtypeStruct((n, f3), jnp.float32),
        a3[0, :n], a3[1, :n], y3[:n], dinv, b3.reshape(1, f3))
    return out


# speedup vs baseline: 2.3364x; 1.1506x over previous
import functools

import jax
import jax.numpy as jnp
from jax import lax
from jax.experimental import pallas as pl
from jax.experimental.pallas import tpu as pltpu
from jax.experimental.pallas import tpu_sc as plsc

_NC = 2
_NS = 16
_NW = _NC * _NS
_GRP = 128


def _mesh():
    return plsc.VectorSubcoreMesh(core_axis_name="c", subcore_axis_name="s")


_SC_PARAMS = pltpu.CompilerParams(use_tc_tiling_on_sc=False)


@functools.lru_cache(maxsize=None)
def _make_deg(n, n_pad, g_per_tile):
    f = 8
    rows_slice = n_pad // _NS

    @functools.partial(
        pl.kernel,
        mesh=_mesh(),
        out_type=jax.ShapeDtypeStruct((_NC, n_pad, f), jnp.float32),
        compiler_params=_SC_PARAMS,
        scratch_types=[
            pltpu.VMEM_SHARED((n_pad, f), jnp.float32),
            pltpu.VMEM((g_per_tile, _GRP), jnp.int32),
            pltpu.VMEM((_GRP, f), jnp.float32),
            pltpu.SemaphoreType.DMA,
        ],
    )
    def deg_kernel(dst_hbm, ones_hbm, zeros_hbm, out_hbm,
                   acc_s, dst_v, ones_v, sem):
        c = lax.axis_index("c")
        s = lax.axis_index("s")
        wid = c * _NS + s

        pltpu.sync_copy(ones_hbm, ones_v)
        pltpu.sync_copy(zeros_hbm,
                        acc_s.at[pl.ds(s * rows_slice, rows_slice)])
        plsc.subcore_barrier()

        pltpu.sync_copy(dst_hbm.at[wid], dst_v)

        def body(j, _):
            cps = [pltpu.async_copy(ones_v, acc_s.at[dst_v.at[4 * j + k]],
                                    sem, add=True) for k in range(4)]
            for cp in cps:
                cp.wait()
            return 0

        lax.fori_loop(0, g_per_tile // 4, body, 0)
        plsc.subcore_barrier()
        pltpu.sync_copy(
            acc_s.at[pl.ds(s * rows_slice, rows_slice)],
            out_hbm.at[c, pl.ds(s * rows_slice, rows_slice)],
        )

    return deg_kernel


@functools.lru_cache(maxsize=None)
def _make_agg(n, n_pad, g_per_tile, f):
    rows_slice = n_pad // _NS

    @functools.partial(
        pl.kernel,
        mesh=_mesh(),
        out_type=jax.ShapeDtypeStruct((_NC, n_pad, f), jnp.float32),
        compiler_params=_SC_PARAMS,
        scratch_types=[
            pltpu.VMEM_SHARED((n_pad, f), jnp.float32),
            pltpu.VMEM_SHARED((n_pad, f), jnp.float32),
            pltpu.VMEM((g_per_tile, _GRP), jnp.int32),
            pltpu.VMEM((g_per_tile, _GRP), jnp.int32),
            pltpu.VMEM((_GRP, f), jnp.float32),
            pltpu.VMEM((_GRP, f), jnp.float32),
            pltpu.VMEM((rows_slice, f), jnp.float32),
            pltpu.SemaphoreType.DMA,
            pltpu.SemaphoreType.DMA,
            pltpu.SemaphoreType.DMA,
            pltpu.SemaphoreType.DMA,
        ],
    )
    def agg_kernel(y_hbm, src_hbm, dst_hbm, out_hbm,
                   y_s, acc_s, src_v, dst_v, rows_a, rows_b, tmp_v,
                   gsem_a, gsem_b, ssem_a, ssem_b):
        c = lax.axis_index("c")
        s = lax.axis_index("s")
        wid = c * _NS + s

        def fill(i, _):
            for j in range(f // 16):
                tmp_v[i, pl.ds(j * 16, 16)] = jnp.zeros((16,), jnp.float32)
            return 0

        lax.fori_loop(0, rows_slice, fill, 0)
        pltpu.sync_copy(tmp_v, acc_s.at[pl.ds(s * rows_slice, rows_slice)])
        pltpu.sync_copy(y_hbm.at[pl.ds(s * rows_slice, rows_slice)],
                        y_s.at[pl.ds(s * rows_slice, rows_slice)])
        pltpu.sync_copy(src_hbm.at[wid], src_v)
        pltpu.sync_copy(dst_hbm.at[wid], dst_v)
        plsc.subcore_barrier()

        n_pairs = g_per_tile // 2

        def gather(g, buf, sem):
            return pltpu.async_copy(y_s.at[src_v.at[g]], buf, sem)

        def scatter(g, buf, sem):
            return pltpu.async_copy(buf, acc_s.at[dst_v.at[g]], sem, add=True)

        gather(0, rows_a, gsem_a)

        def body(j, _):
            g0 = 2 * j
            pltpu.make_async_copy(y_s.at[src_v.at[g0]], rows_a, gsem_a).wait()

            @pl.when(j > 0)
            def _():
                pltpu.make_async_copy(rows_b, acc_s.at[dst_v.at[g0]],
                                      ssem_b).wait()

            gather(g0 + 1, rows_b, gsem_b)
            scatter(g0, rows_a, ssem_a)
            pltpu.make_async_copy(y_s.at[src_v.at[g0]], rows_b, gsem_b).wait()
            pltpu.make_async_copy(rows_a, acc_s.at[dst_v.at[g0]],
                                  ssem_a).wait()

            @pl.when(j + 1 < n_pairs)
            def _():
                gather(g0 + 2, rows_a, gsem_a)

            scatter(g0 + 1, rows_b, ssem_b)
            return 0

        lax.fori_loop(0, n_pairs, body, 0)
        pltpu.make_async_copy(rows_b, acc_s.at[dst_v.at[0]], ssem_b).wait()
        plsc.subcore_barrier()
        pltpu.sync_copy(
            acc_s.at[pl.ds(s * rows_slice, rows_slice)],
            out_hbm.at[c, pl.ds(s * rows_slice, rows_slice)],
        )

    return agg_kernel


def _tc_call(body, out_shapes, *args):
    return pl.pallas_call(body, out_shape=out_shapes)(*args)


def _tc1_body(x_ref, w1_ref, degp_ref, y1_ref, dinv_ref):
    n = x_ref.shape[0]
    pad = y1_ref.shape[0] - n
    deg = degp_ref[0, :n, :1] + degp_ref[1, :n, :1] + 1.0
    dinv = lax.rsqrt(deg)
    hw = jnp.dot(x_ref[...], w1_ref[...], preferred_element_type=jnp.float32)
    y1_ref[:n, :] = hw * dinv
    y1_ref[n:, :] = jnp.zeros((pad, y1_ref.shape[1]), jnp.float32)
    dinv_ref[...] = dinv


def _tc_mid_body(aggp_ref, y_ref, dinv_ref, b_ref, g_ref, be_ref,
                 w_ref, out_ref):
    n = dinv_ref.shape[0]
    dinv = dinv_ref[...]
    z = (aggp_ref[0, :n, :] + aggp_ref[1, :n, :] + y_ref[:n, :]) * dinv
    z = z + b_ref[...]
    mu = jnp.mean(z, axis=0, keepdims=True)
    var = jnp.mean((z - mu) ** 2, axis=0, keepdims=True)
    h = (z - mu) * lax.rsqrt(var + 1e-5) * g_ref[...] + be_ref[...]
    h = jax.nn.gelu(h)
    pad = out_ref.shape[0] - n
    out_ref[:n, :] = jnp.dot(h, w_ref[...],
                             preferred_element_type=jnp.float32) * dinv
    out_ref[n:, :] = jnp.zeros((pad, out_ref.shape[1]), jnp.float32)


def _tc_out_body(aggp_ref, y_ref, dinv_ref, b_ref, out_ref):
    n = dinv_ref.shape[0]
    z = (aggp_ref[0, :n, :] + aggp_ref[1, :n, :] + y_ref[:n, :])
    z = z * dinv_ref[...] + b_ref[...]
    m = jnp.max(z, axis=1, keepdims=True)
    sh = z - m
    out_ref[...] = sh - jnp.log(jnp.sum(jnp.exp(sh), axis=1, keepdims=True))


def kernel(x, edge_index, W1, b1, g1, be1, W2, b2, g2, be2, W3, b3):
    n, d = x.shape
    e = edge_index.shape[1]
    f1, f2, f3 = W1.shape[1], W2.shape[1], W3.shape[1]

    n_pad = (n + 1 + 127) // 128 * 128
    per_tile = (e + _NW - 1) // _NW
    g_per_tile = (per_tile + _GRP - 1) // _GRP
    g_per_tile = (g_per_tile + 3) // 4 * 4
    cap = _NW * g_per_tile * _GRP

    src = edge_index[0]
    dst = edge_index[1]
    pad = cap - e
    src_p = jnp.concatenate([src, jnp.zeros((pad,), jnp.int32)])
    dst_p = jnp.concatenate([dst, jnp.full((pad,), n, jnp.int32)])
    src_r = src_p.reshape(_NW, g_per_tile, _GRP)
    dst_r = dst_p.reshape(_NW, g_per_tile, _GRP)

    ones8 = jnp.ones((_GRP, 8), jnp.float32)
    zeros8 = jnp.zeros((n_pad // _NS, 8), jnp.float32)
    degp = _make_deg(n, n_pad, g_per_tile)(dst_r, ones8, zeros8)
    y1, dinv = _tc_call(
        _tc1_body,
        [jax.ShapeDtypeStruct((n_pad, f1), jnp.float32),
         jax.ShapeDtypeStruct((n, 1), jnp.float32)],
        x, W1, degp)

    agg16 = _make_agg(n, n_pad, g_per_tile, f1)
    agg32 = _make_agg(n, n_pad, g_per_tile, f2)

    a1 = agg16(y1, src_r, dst_r)
    y2 = _tc_call(
        _tc_mid_body, jax.ShapeDtypeStruct((n_pad, f2), jnp.float32),
        a1, y1, dinv,
        b1.reshape(1, f1), g1.reshape(1, f1), be1.reshape(1, f1), W2)

    a2 = agg32(y2, src_r, dst_r)
    y3 = _tc_call(
        _tc_mid_body, jax.ShapeDtypeStruct((n_pad, f3), jnp.float32),
        a2, y2, dinv,
        b2.reshape(1, f2), g2.reshape(1, f2), be2.reshape(1, f2), W3)

    a3 = _make_agg(n, n_pad, g_per_tile, f3)(y3, src_r, dst_r)
    out = _tc_call(
        _tc_out_body, jax.ShapeDtypeStruct((n, f3), jnp.float32),
        a3, y3, dinv, b3.reshape(1, f3))
    return out


# speedup vs baseline: 2.3494x; 1.0056x over previous
import functools

import jax
import jax.numpy as jnp
from jax import lax
from jax.experimental import pallas as pl
from jax.experimental.pallas import tpu as pltpu
from jax.experimental.pallas import tpu_sc as plsc

_NC = 2
_NS = 16
_NW = _NC * _NS
_GRP = 256


def _mesh():
    return plsc.VectorSubcoreMesh(core_axis_name="c", subcore_axis_name="s")


_SC_PARAMS = pltpu.CompilerParams(use_tc_tiling_on_sc=False)


@functools.lru_cache(maxsize=None)
def _make_deg(n, n_pad, g_per_tile):
    f = 8
    rows_slice = n_pad // _NS

    @functools.partial(
        pl.kernel,
        mesh=_mesh(),
        out_type=jax.ShapeDtypeStruct((_NC, n_pad, f), jnp.float32),
        compiler_params=_SC_PARAMS,
        scratch_types=[
            pltpu.VMEM_SHARED((n_pad, f), jnp.float32),
            pltpu.VMEM((g_per_tile, _GRP), jnp.int32),
            pltpu.VMEM((_GRP, f), jnp.float32),
            pltpu.SemaphoreType.DMA,
        ],
    )
    def deg_kernel(dst_hbm, ones_hbm, zeros_hbm, out_hbm,
                   acc_s, dst_v, ones_v, sem):
        c = lax.axis_index("c")
        s = lax.axis_index("s")
        wid = c * _NS + s

        pltpu.sync_copy(ones_hbm, ones_v)
        pltpu.sync_copy(zeros_hbm,
                        acc_s.at[pl.ds(s * rows_slice, rows_slice)])
        plsc.subcore_barrier()

        pltpu.sync_copy(dst_hbm.at[wid], dst_v)

        def body(j, _):
            cps = [pltpu.async_copy(ones_v, acc_s.at[dst_v.at[4 * j + k]],
                                    sem, add=True) for k in range(4)]
            for cp in cps:
                cp.wait()
            return 0

        lax.fori_loop(0, g_per_tile // 4, body, 0)
        plsc.subcore_barrier()
        pltpu.sync_copy(
            acc_s.at[pl.ds(s * rows_slice, rows_slice)],
            out_hbm.at[c, pl.ds(s * rows_slice, rows_slice)],
        )

    return deg_kernel


@functools.lru_cache(maxsize=None)
def _make_agg(n, n_pad, g_per_tile, f):
    rows_slice = n_pad // _NS

    @functools.partial(
        pl.kernel,
        mesh=_mesh(),
        out_type=jax.ShapeDtypeStruct((_NC, n_pad, f), jnp.float32),
        compiler_params=_SC_PARAMS,
        scratch_types=[
            pltpu.VMEM_SHARED((n_pad, f), jnp.float32),
            pltpu.VMEM_SHARED((n_pad, f), jnp.float32),
            pltpu.VMEM((g_per_tile, _GRP), jnp.int32),
            pltpu.VMEM((g_per_tile, _GRP), jnp.int32),
            pltpu.VMEM((_GRP, f), jnp.float32),
            pltpu.VMEM((_GRP, f), jnp.float32),
            pltpu.VMEM((rows_slice, f), jnp.float32),
            pltpu.SemaphoreType.DMA,
            pltpu.SemaphoreType.DMA,
            pltpu.SemaphoreType.DMA,
            pltpu.SemaphoreType.DMA,
        ],
    )
    def agg_kernel(y_hbm, src_hbm, dst_hbm, out_hbm,
                   y_s, acc_s, src_v, dst_v, rows_a, rows_b, tmp_v,
                   gsem_a, gsem_b, ssem_a, ssem_b):
        c = lax.axis_index("c")
        s = lax.axis_index("s")
        wid = c * _NS + s

        def fill(i, _):
            for j in range(f // 16):
                tmp_v[i, pl.ds(j * 16, 16)] = jnp.zeros((16,), jnp.float32)
            return 0

        lax.fori_loop(0, rows_slice, fill, 0)
        pltpu.sync_copy(tmp_v, acc_s.at[pl.ds(s * rows_slice, rows_slice)])
        pltpu.sync_copy(y_hbm.at[pl.ds(s * rows_slice, rows_slice)],
                        y_s.at[pl.ds(s * rows_slice, rows_slice)])
        pltpu.sync_copy(src_hbm.at[wid], src_v)
        pltpu.sync_copy(dst_hbm.at[wid], dst_v)
        plsc.subcore_barrier()

        n_pairs = g_per_tile // 2

        def gather(g, buf, sem):
            return pltpu.async_copy(y_s.at[src_v.at[g]], buf, sem)

        def scatter(g, buf, sem):
            return pltpu.async_copy(buf, acc_s.at[dst_v.at[g]], sem, add=True)

        gather(0, rows_a, gsem_a)

        def body(j, _):
            g0 = 2 * j
            pltpu.make_async_copy(y_s.at[src_v.at[g0]], rows_a, gsem_a).wait()

            @pl.when(j > 0)
            def _():
                pltpu.make_async_copy(rows_b, acc_s.at[dst_v.at[g0]],
                                      ssem_b).wait()

            gather(g0 + 1, rows_b, gsem_b)
            scatter(g0, rows_a, ssem_a)
            pltpu.make_async_copy(y_s.at[src_v.at[g0]], rows_b, gsem_b).wait()
            pltpu.make_async_copy(rows_a, acc_s.at[dst_v.at[g0]],
                                  ssem_a).wait()

            @pl.when(j + 1 < n_pairs)
            def _():
                gather(g0 + 2, rows_a, gsem_a)

            scatter(g0 + 1, rows_b, ssem_b)
            return 0

        lax.fori_loop(0, n_pairs, body, 0)
        pltpu.make_async_copy(rows_b, acc_s.at[dst_v.at[0]], ssem_b).wait()
        plsc.subcore_barrier()
        pltpu.sync_copy(
            acc_s.at[pl.ds(s * rows_slice, rows_slice)],
            out_hbm.at[c, pl.ds(s * rows_slice, rows_slice)],
        )

    return agg_kernel


def _tc_call(body, out_shapes, *args):
    return pl.pallas_call(body, out_shape=out_shapes)(*args)


def _tc1_body(x_ref, w1_ref, degp_ref, y1_ref, dinv_ref):
    n = x_ref.shape[0]
    pad = y1_ref.shape[0] - n
    deg = degp_ref[0, :n, :1] + degp_ref[1, :n, :1] + 1.0
    dinv = lax.rsqrt(deg)
    hw = jnp.dot(x_ref[...], w1_ref[...], preferred_element_type=jnp.float32)
    y1_ref[:n, :] = hw * dinv
    y1_ref[n:, :] = jnp.zeros((pad, y1_ref.shape[1]), jnp.float32)
    dinv_ref[...] = dinv


def _tc_mid_body(aggp_ref, y_ref, dinv_ref, b_ref, g_ref, be_ref,
                 w_ref, out_ref):
    n = dinv_ref.shape[0]
    dinv = dinv_ref[...]
    z = (aggp_ref[0, :n, :] + aggp_ref[1, :n, :] + y_ref[:n, :]) * dinv
    z = z + b_ref[...]
    mu = jnp.mean(z, axis=0, keepdims=True)
    var = jnp.mean((z - mu) ** 2, axis=0, keepdims=True)
    h = (z - mu) * lax.rsqrt(var + 1e-5) * g_ref[...] + be_ref[...]
    h = jax.nn.gelu(h)
    pad = out_ref.shape[0] - n
    out_ref[:n, :] = jnp.dot(h, w_ref[...],
                             preferred_element_type=jnp.float32) * dinv
    out_ref[n:, :] = jnp.zeros((pad, out_ref.shape[1]), jnp.float32)


def _tc_out_body(aggp_ref, y_ref, dinv_ref, b_ref, out_ref):
    n = dinv_ref.shape[0]
    z = (aggp_ref[0, :n, :] + aggp_ref[1, :n, :] + y_ref[:n, :])
    z = z * dinv_ref[...] + b_ref[...]
    m = jnp.max(z, axis=1, keepdims=True)
    sh = z - m
    out_ref[...] = sh - jnp.log(jnp.sum(jnp.exp(sh), axis=1, keepdims=True))


def kernel(x, edge_index, W1, b1, g1, be1, W2, b2, g2, be2, W3, b3):
    n, d = x.shape
    e = edge_index.shape[1]
    f1, f2, f3 = W1.shape[1], W2.shape[1], W3.shape[1]

    n_pad = (n + 1 + 127) // 128 * 128
    per_tile = (e + _NW - 1) // _NW
    g_per_tile = (per_tile + _GRP - 1) // _GRP
    g_per_tile = (g_per_tile + 3) // 4 * 4
    cap = _NW * g_per_tile * _GRP

    src = edge_index[0]
    dst = edge_index[1]
    pad = cap - e
    src_p = jnp.concatenate([src, jnp.zeros((pad,), jnp.int32)])
    dst_p = jnp.concatenate([dst, jnp.full((pad,), n, jnp.int32)])
    src_r = src_p.reshape(_NW, g_per_tile, _GRP)
    dst_r = dst_p.reshape(_NW, g_per_tile, _GRP)

    ones8 = jnp.ones((_GRP, 8), jnp.float32)
    zeros8 = jnp.zeros((n_pad // _NS, 8), jnp.float32)
    degp = _make_deg(n, n_pad, g_per_tile)(dst_r, ones8, zeros8)
    y1, dinv = _tc_call(
        _tc1_body,
        [jax.ShapeDtypeStruct((n_pad, f1), jnp.float32),
         jax.ShapeDtypeStruct((n, 1), jnp.float32)],
        x, W1, degp)

    agg16 = _make_agg(n, n_pad, g_per_tile, f1)
    agg32 = _make_agg(n, n_pad, g_per_tile, f2)

    a1 = agg16(y1, src_r, dst_r)
    y2 = _tc_call(
        _tc_mid_body, jax.ShapeDtypeStruct((n_pad, f2), jnp.float32),
        a1, y1, dinv,
        b1.reshape(1, f1), g1.reshape(1, f1), be1.reshape(1, f1), W2)

    a2 = agg32(y2, src_r, dst_r)
    y3 = _tc_call(
        _tc_mid_body, jax.ShapeDtypeStruct((n_pad, f3), jnp.float32),
        a2, y2, dinv,
        b2.reshape(1, f2), g2.reshape(1, f2), be2.reshape(1, f2), W3)

    a3 = _make_agg(n, n_pad, g_per_tile, f3)(y3, src_r, dst_r)
    out = _tc_call(
        _tc_out_body, jax.ShapeDtypeStruct((n, f3), jnp.float32),
        a3, y3, dinv, b3.reshape(1, f3))
    return out


# speedup vs baseline: 2.4117x; 1.0265x over previous
import functools

import jax
import jax.numpy as jnp
from jax import lax
from jax.experimental import pallas as pl
from jax.experimental.pallas import tpu as pltpu
from jax.experimental.pallas import tpu_sc as plsc

_NC = 2
_NS = 16
_NW = _NC * _NS
_GRP = 250


def _mesh():
    return plsc.VectorSubcoreMesh(core_axis_name="c", subcore_axis_name="s")


_SC_PARAMS = pltpu.CompilerParams(use_tc_tiling_on_sc=False)


@functools.lru_cache(maxsize=None)
def _make_deg(n, n_pad, g_per_tile):
    f = 8
    rows_slice = n_pad // _NS

    @functools.partial(
        pl.kernel,
        mesh=_mesh(),
        out_type=jax.ShapeDtypeStruct((_NC, n_pad, f), jnp.float32),
        compiler_params=_SC_PARAMS,
        scratch_types=[
            pltpu.VMEM_SHARED((n_pad, f), jnp.float32),
            pltpu.VMEM((g_per_tile, _GRP), jnp.int32),
            pltpu.VMEM((_GRP, f), jnp.float32),
            pltpu.SemaphoreType.DMA,
        ],
    )
    def deg_kernel(dst_hbm, ones_hbm, zeros_hbm, out_hbm,
                   acc_s, dst_v, ones_v, sem):
        c = lax.axis_index("c")
        s = lax.axis_index("s")
        wid = c * _NS + s

        pltpu.sync_copy(ones_hbm, ones_v)
        pltpu.sync_copy(zeros_hbm,
                        acc_s.at[pl.ds(s * rows_slice, rows_slice)])
        plsc.subcore_barrier()

        pltpu.sync_copy(dst_hbm.at[wid], dst_v)

        def body(j, _):
            cps = [pltpu.async_copy(ones_v, acc_s.at[dst_v.at[4 * j + k]],
                                    sem, add=True) for k in range(4)]
            for cp in cps:
                cp.wait()
            return 0

        lax.fori_loop(0, g_per_tile // 4, body, 0)
        plsc.subcore_barrier()
        pltpu.sync_copy(
            acc_s.at[pl.ds(s * rows_slice, rows_slice)],
            out_hbm.at[c, pl.ds(s * rows_slice, rows_slice)],
        )

    return deg_kernel


@functools.lru_cache(maxsize=None)
def _make_agg(n, n_pad, g_per_tile, f):
    rows_slice = n_pad // _NS

    @functools.partial(
        pl.kernel,
        mesh=_mesh(),
        out_type=jax.ShapeDtypeStruct((_NC, n_pad, f), jnp.float32),
        compiler_params=_SC_PARAMS,
        scratch_types=[
            pltpu.VMEM_SHARED((n_pad, f), jnp.float32),
            pltpu.VMEM_SHARED((n_pad, f), jnp.float32),
            pltpu.VMEM((g_per_tile, _GRP), jnp.int32),
            pltpu.VMEM((g_per_tile, _GRP), jnp.int32),
            pltpu.VMEM((_GRP, f), jnp.float32),
            pltpu.VMEM((_GRP, f), jnp.float32),
            pltpu.VMEM((rows_slice, f), jnp.float32),
            pltpu.SemaphoreType.DMA,
            pltpu.SemaphoreType.DMA,
            pltpu.SemaphoreType.DMA,
            pltpu.SemaphoreType.DMA,
        ],
    )
    def agg_kernel(y_hbm, src_hbm, dst_hbm, out_hbm,
                   y_s, acc_s, src_v, dst_v, rows_a, rows_b, tmp_v,
                   gsem_a, gsem_b, ssem_a, ssem_b):
        c = lax.axis_index("c")
        s = lax.axis_index("s")
        wid = c * _NS + s

        def fill(i, _):
            for j in range(f // 16):
                tmp_v[i, pl.ds(j * 16, 16)] = jnp.zeros((16,), jnp.float32)
            return 0

        lax.fori_loop(0, rows_slice, fill, 0)
        pltpu.sync_copy(tmp_v, acc_s.at[pl.ds(s * rows_slice, rows_slice)])
        pltpu.sync_copy(y_hbm.at[pl.ds(s * rows_slice, rows_slice)],
                        y_s.at[pl.ds(s * rows_slice, rows_slice)])
        pltpu.sync_copy(src_hbm.at[wid], src_v)
        pltpu.sync_copy(dst_hbm.at[wid], dst_v)
        plsc.subcore_barrier()

        n_pairs = g_per_tile // 2

        def gather(g, buf, sem):
            return pltpu.async_copy(y_s.at[src_v.at[g]], buf, sem)

        def scatter(g, buf, sem):
            return pltpu.async_copy(buf, acc_s.at[dst_v.at[g]], sem, add=True)

        gather(0, rows_a, gsem_a)

        def body(j, _):
            g0 = 2 * j
            pltpu.make_async_copy(y_s.at[src_v.at[g0]], rows_a, gsem_a).wait()

            @pl.when(j > 0)
            def _():
                pltpu.make_async_copy(rows_b, acc_s.at[dst_v.at[g0]],
                                      ssem_b).wait()

            gather(g0 + 1, rows_b, gsem_b)
            scatter(g0, rows_a, ssem_a)
            pltpu.make_async_copy(y_s.at[src_v.at[g0]], rows_b, gsem_b).wait()
            pltpu.make_async_copy(rows_a, acc_s.at[dst_v.at[g0]],
                                  ssem_a).wait()

            @pl.when(j + 1 < n_pairs)
            def _():
                gather(g0 + 2, rows_a, gsem_a)

            scatter(g0 + 1, rows_b, ssem_b)
            return 0

        lax.fori_loop(0, n_pairs, body, 0)
        pltpu.make_async_copy(rows_b, acc_s.at[dst_v.at[0]], ssem_b).wait()
        plsc.subcore_barrier()
        pltpu.sync_copy(
            acc_s.at[pl.ds(s * rows_slice, rows_slice)],
            out_hbm.at[c, pl.ds(s * rows_slice, rows_slice)],
        )

    return agg_kernel


def _tc_call(body, out_shapes, *args):
    return pl.pallas_call(body, out_shape=out_shapes)(*args)


def _tc1_body(x_ref, w1_ref, degp_ref, y1_ref, dinv_ref):
    n = x_ref.shape[0]
    pad = y1_ref.shape[0] - n
    deg = degp_ref[0, :n, :1] + degp_ref[1, :n, :1] + 1.0
    dinv = lax.rsqrt(deg)
    hw = jnp.dot(x_ref[...], w1_ref[...], preferred_element_type=jnp.float32)
    y1_ref[:n, :] = hw * dinv
    y1_ref[n:, :] = jnp.zeros((pad, y1_ref.shape[1]), jnp.float32)
    dinv_ref[...] = dinv


def _tc_mid_body(aggp_ref, y_ref, dinv_ref, b_ref, g_ref, be_ref,
                 w_ref, out_ref):
    n = dinv_ref.shape[0]
    dinv = dinv_ref[...]
    z = (aggp_ref[0, :n, :] + aggp_ref[1, :n, :] + y_ref[:n, :]) * dinv
    z = z + b_ref[...]
    mu = jnp.mean(z, axis=0, keepdims=True)
    var = jnp.mean((z - mu) ** 2, axis=0, keepdims=True)
    h = (z - mu) * lax.rsqrt(var + 1e-5) * g_ref[...] + be_ref[...]
    h = jax.nn.gelu(h)
    pad = out_ref.shape[0] - n
    out_ref[:n, :] = jnp.dot(h, w_ref[...],
                             preferred_element_type=jnp.float32) * dinv
    out_ref[n:, :] = jnp.zeros((pad, out_ref.shape[1]), jnp.float32)


def _tc_out_body(aggp_ref, y_ref, dinv_ref, b_ref, out_ref):
    n = dinv_ref.shape[0]
    z = (aggp_ref[0, :n, :] + aggp_ref[1, :n, :] + y_ref[:n, :])
    z = z * dinv_ref[...] + b_ref[...]
    m = jnp.max(z, axis=1, keepdims=True)
    sh = z - m
    out_ref[...] = sh - jnp.log(jnp.sum(jnp.exp(sh), axis=1, keepdims=True))


def kernel(x, edge_index, W1, b1, g1, be1, W2, b2, g2, be2, W3, b3):
    n, d = x.shape
    e = edge_index.shape[1]
    f1, f2, f3 = W1.shape[1], W2.shape[1], W3.shape[1]

    n_pad = (n + 1 + 127) // 128 * 128
    g_per_tile = (e + _NW * _GRP - 1) // (_NW * _GRP)
    g_per_tile = (g_per_tile + 3) // 4 * 4
    cap = _NW * g_per_tile * _GRP

    src = edge_index[0]
    dst = edge_index[1]
    pad = cap - e
    if pad:
        src = jnp.concatenate([src, jnp.zeros((pad,), jnp.int32)])
        dst = jnp.concatenate([dst, jnp.full((pad,), n, jnp.int32)])
    src_r = src.reshape(_NW, g_per_tile, _GRP)
    dst_r = dst.reshape(_NW, g_per_tile, _GRP)

    ones8 = jnp.ones((_GRP, 8), jnp.float32)
    zeros8 = jnp.zeros((n_pad // _NS, 8), jnp.float32)
    degp = _make_deg(n, n_pad, g_per_tile)(dst_r, ones8, zeros8)
    y1, dinv = _tc_call(
        _tc1_body,
        [jax.ShapeDtypeStruct((n_pad, f1), jnp.float32),
         jax.ShapeDtypeStruct((n, 1), jnp.float32)],
        x, W1, degp)

    agg16 = _make_agg(n, n_pad, g_per_tile, f1)
    agg32 = _make_agg(n, n_pad, g_per_tile, f2)

    a1 = agg16(y1, src_r, dst_r)
    y2 = _tc_call(
        _tc_mid_body, jax.ShapeDtypeStruct((n_pad, f2), jnp.float32),
        a1, y1, dinv,
        b1.reshape(1, f1), g1.reshape(1, f1), be1.reshape(1, f1), W2)

    a2 = agg32(y2, src_r, dst_r)
    y3 = _tc_call(
        _tc_mid_body, jax.ShapeDtypeStruct((n_pad, f3), jnp.float32),
        a2, y2, dinv,
        b2.reshape(1, f2), g2.reshape(1, f2), be2.reshape(1, f2), W3)

    a3 = _make_agg(n, n_pad, g_per_tile, f3)(y3, src_r, dst_r)
    out = _tc_call(
        _tc_out_body, jax.ShapeDtypeStruct((n, f3), jnp.float32),
        a3, y3, dinv, b3.reshape(1, f3))
    return out
